# Initial kernel scaffold; baseline (speedup 1.0000x reference)
import jax
import jax.numpy as jnp
from jax.experimental import pallas as pl


def kernel(x, edge_index, W1, b1, W2, b2):
    raise NotImplementedError("write your pallas kernel here")


# speedup vs baseline: 10.8633x; 10.8633x over previous
import functools

import jax
import jax.numpy as jnp
from jax import lax
from jax.experimental import pallas as pl
from jax.experimental.pallas import tpu as pltpu
from jax.experimental.pallas import tpu_sc as plsc

N_NODES = 10000
N_EDGES = 320000
D_FEAT = 128
D_PAD = 128
D_OUT = 64

NC = 2
NS = 16
LANES = 16

CHUNK = 128
ROWS_PER_TILE = 640
NP = NS * ROWS_PER_TILE
EP = 323584
CHUNKS_PER_TILE = EP // (NC * NS * CHUNK)

_f32 = jnp.float32
_i32 = jnp.int32


def _fill_vec(ref, n16, value):
    v = jnp.full((LANES,), value, dtype=_f32)
    for i in range(n16):
        ref[pl.ds(i * LANES, LANES)] = v


def _zero_rows(ref):
    z = jnp.zeros((LANES,), dtype=_f32)

    def body(i, _):
        for j in range(D_PAD // LANES):
            ref[i, pl.ds(j * LANES, LANES)] = z
        return 0

    lax.fori_loop(0, CHUNK, body, 0)


def _sc_mesh():
    return plsc.VectorSubcoreMesh(
        core_axis_name="c", subcore_axis_name="s", num_cores=NC, num_subcores=NS
    )


@functools.partial(
    pl.kernel,
    out_type=(jax.ShapeDtypeStruct((NP,), _f32), jax.ShapeDtypeStruct((NP,), _f32)),
    mesh=_sc_mesh(),
    scratch_types=[
        pltpu.VMEM((CHUNK,), _i32),
        pltpu.VMEM((CHUNK,), _f32),
        pltpu.VMEM_SHARED((NP,), _f32)
    ],
)
def _sc_degree(dst_hbm, out0_hbm, out1_hbm, idx_d, val_v, dacc):
    c = lax.axis_index("c")
    s = lax.axis_index("s")
    t = c * NS + s
    row0 = s * ROWS_PER_TILE

    _fill_vec(val_v, CHUNK // LANES, 0.0)
    for k in range(ROWS_PER_TILE // CHUNK):
        pltpu.sync_copy(val_v, dacc.at[pl.ds(row0 + k * CHUNK, CHUNK)])
    _fill_vec(val_v, CHUNK // LANES, 1.0)
    plsc.subcore_barrier()

    def body(j, _):
        base = (t * CHUNKS_PER_TILE + j) * CHUNK
        pltpu.sync_copy(dst_hbm.at[pl.ds(base, CHUNK)], idx_d)
        pltpu.sync_copy(val_v, dacc.at[idx_d], add=True)
        return 0

    lax.fori_loop(0, CHUNKS_PER_TILE, body, 0)
    plsc.subcore_barrier()

    for k in range(ROWS_PER_TILE // CHUNK):
        sl = pl.ds(row0 + k * CHUNK, CHUNK)
        pltpu.sync_copy(dacc.at[sl], val_v)

        @pl.when(c == 0)
        def _():
            pltpu.sync_copy(val_v, out0_hbm.at[sl])

        @pl.when(c == 1)
        def _():
            pltpu.sync_copy(val_v, out1_hbm.at[sl])


@functools.partial(
    pl.kernel,
    out_type=jax.ShapeDtypeStruct((NC, NP, D_PAD), _f32),
    mesh=_sc_mesh(),
    scratch_types=[
        pltpu.VMEM((CHUNK,), _i32),
        pltpu.VMEM((CHUNK,), _i32),
        pltpu.VMEM((CHUNK, D_PAD), _f32),
        pltpu.SemaphoreType.DMA,
        pltpu.VMEM_SHARED((NP, D_PAD), _f32),
    ],
)
def _sc_scatter(g_hbm, src_hbm, dst_hbm, out_hbm, idx_s, idx_d, rows, sem, acc):
    c = lax.axis_index("c")
    s = lax.axis_index("s")
    t = c * NS + s
    row0 = s * ROWS_PER_TILE

    _zero_rows(rows)
    for k in range(ROWS_PER_TILE // CHUNK):
        sl = pl.ds(row0 + k * CHUNK, CHUNK)
        pltpu.sync_copy(rows, acc.at[sl])
    plsc.subcore_barrier()

    def body(j, _):
        base = (t * CHUNKS_PER_TILE + j) * CHUNK
        pltpu.sync_copy(src_hbm.at[pl.ds(base, CHUNK)], idx_s)
        pltpu.sync_copy(dst_hbm.at[pl.ds(base, CHUNK)], idx_d)
        pltpu.async_copy(g_hbm.at[idx_s], rows, sem).wait()
        pltpu.sync_copy(rows, acc.at[idx_d], add=True)
        return 0

    lax.fori_loop(0, CHUNKS_PER_TILE, body, 0)
    plsc.subcore_barrier()

    for k in range(ROWS_PER_TILE // CHUNK):
        sl = pl.ds(row0 + k * CHUNK, CHUNK)
        pltpu.sync_copy(acc.at[sl], rows)
        pltpu.sync_copy(rows, out_hbm.at[c, sl])


_BLK1 = 2000


def _dis_col(deg_ref):
    d = deg_ref[:, 0] + deg_ref[:, 1] + 1.0
    return lax.rsqrt(d)[:, None]


def _tc1_body(x_ref, w_ref, b_ref, deg_ref, g_ref):
    h = jnp.dot(x_ref[...], w_ref[...], preferred_element_type=_f32) + b_ref[...]
    g_ref[...] = _dis_col(deg_ref) * h


def _tc2_body(deg_ref, s_ref, g_ref, w_ref, b_ref, out_ref):
    dis = _dis_col(deg_ref)
    z = dis * (s_ref[0] + s_ref[1] + g_ref[...])
    a = jnp.maximum(z, 0.0)
    h2 = jnp.dot(a, w_ref[...], preferred_element_type=_f32) + b_ref[...]
    out_ref[...] = dis * h2


def _tc3_body(deg_ref, s_ref, g_ref, out_ref):
    zf = _dis_col(deg_ref) * (s_ref[0] + s_ref[1] + g_ref[...])
    z = zf[:, :D_OUT]
    m = jnp.max(z, axis=1, keepdims=True)
    lse = jnp.log(jnp.sum(jnp.exp(z - m), axis=1, keepdims=True)) + m
    out_ref[...] = z - lse


def kernel(x, edge_index, W1, b1, W2, b2):
    src = edge_index[0].astype(_i32)
    dst = edge_index[1].astype(_i32)
    pad = jnp.full((EP - N_EDGES,), N_NODES, dtype=_i32)
    src_p = jnp.concatenate([src, pad])
    dst_p = jnp.concatenate([dst, pad])

    w1t = jnp.zeros((D_FEAT, D_PAD), _f32).at[:, : W1.shape[0]].set(W1.T)
    b1p = jnp.zeros((1, D_PAD), _f32).at[0, : b1.shape[0]].set(b1)
    w2t = jnp.zeros((D_PAD, D_PAD), _f32).at[: W2.shape[1], : W2.shape[0]].set(W2.T)
    b2p = jnp.zeros((1, D_PAD), _f32).at[0, : b2.shape[0]].set(b2)

    deg0, deg1 = _sc_degree(dst_p)
    degp = jnp.stack([deg0, deg1], axis=-1)

    g1 = pl.pallas_call(
        _tc1_body,
        grid=(N_NODES // _BLK1,),
        in_specs=[
            pl.BlockSpec((_BLK1, D_FEAT), lambda i: (i, 0)),
            pl.BlockSpec((D_FEAT, D_PAD), lambda i: (0, 0)),
            pl.BlockSpec((1, D_PAD), lambda i: (0, 0)),
            pl.BlockSpec((_BLK1, NC), lambda i: (i, 0)),
        ],
        out_specs=pl.BlockSpec((_BLK1, D_PAD), lambda i: (i, 0)),
        out_shape=jax.ShapeDtypeStruct((NP, D_PAD), _f32),
    )(x, w1t, b1p, degp)

    s1 = _sc_scatter(g1, src_p, dst_p)

    g2 = pl.pallas_call(
        _tc2_body,
        grid=(N_NODES // _BLK1,),
        in_specs=[
            pl.BlockSpec((_BLK1, NC), lambda i: (i, 0)),
            pl.BlockSpec((NC, _BLK1, D_PAD), lambda i: (0, i, 0)),
            pl.BlockSpec((_BLK1, D_PAD), lambda i: (i, 0)),
            pl.BlockSpec((D_PAD, D_PAD), lambda i: (0, 0)),
            pl.BlockSpec((1, D_PAD), lambda i: (0, 0)),
        ],
        out_specs=pl.BlockSpec((_BLK1, D_PAD), lambda i: (i, 0)),
        out_shape=jax.ShapeDtypeStruct((NP, D_PAD), _f32),
    )(degp, s1, g1, w2t, b2p)

    s2 = _sc_scatter(g2, src_p, dst_p)

    out = pl.pallas_call(
        _tc3_body,
        grid=(N_NODES // _BLK1,),
        in_specs=[
            pl.BlockSpec((_BLK1, NC), lambda i: (i, 0)),
            pl.BlockSpec((NC, _BLK1, D_PAD), lambda i: (0, i, 0)),
            pl.BlockSpec((_BLK1, D_PAD), lambda i: (i, 0)),
        ],
        out_specs=pl.BlockSpec((_BLK1, D_OUT), lambda i: (i, 0)),
        out_shape=jax.ShapeDtypeStruct((N_NODES, D_OUT), _f32),
    )(degp, s2, g2)

    return out
